# Initial kernel scaffold; baseline (speedup 1.0000x reference)
import jax
import jax.numpy as jnp
from jax.experimental import pallas as pl


def kernel(x, edge_index, doc_word_ids, doc_tfidf, Wl1, Wr1, b1, g1, be1, Wl2, Wr2, b2, g2, be2, Wfc, bfc):
    raise NotImplementedError("write your pallas kernel here")


# speedup vs baseline: 3.2738x; 3.2738x over previous
import dataclasses
import functools

import jax
import jax.numpy as jnp
from jax import lax
from jax.experimental import pallas as pl
from jax.experimental.pallas import tpu as pltpu
from jax.experimental.pallas import tpu_sc as plsc

N = 10000
E = 160000
D = 256
H = 128
NCLS = 16
NDOC = 4096
DLEN = 64

NS = 16
CHUNK = 128
NCHUNK = E // CHUNK
NP = 10240
ROWS_PER_TILE = NP // NS
LAST_ROWS = N - (NS - 1) * ROWS_PER_TILE
DOCS_PER_TILE = NDOC // NS
DB = 8
BR = 1000
EPS_BN = 1e-5
EPS_POOL = 1e-8


def _sc_mesh():
    return plsc.VectorSubcoreMesh(core_axis_name="c", subcore_axis_name="s")


def _sc_params():
    cp = pltpu.CompilerParams()
    if "needs_layout_passes" in pltpu.CompilerParams.__dataclass_fields__:
        cp = dataclasses.replace(cp, needs_layout_passes=False)
    return cp


@functools.partial(
    pl.kernel,
    out_type=[
        jax.ShapeDtypeStruct((N, H), jnp.float32),
        jax.ShapeDtypeStruct((N, H), jnp.float32),
    ],
    mesh=_sc_mesh(),
    scratch_types=[
        pltpu.VMEM_SHARED((NP, H), jnp.float32),
        pltpu.VMEM((2, CHUNK), jnp.int32),
        pltpu.VMEM((CHUNK, H), jnp.float32),
        pltpu.SemaphoreType.DMA,
    ],
    compiler_params=_sc_params(),
)
def _seg_sum(x0_hbm, x1_hbm, src_hbm, dst_hbm, z_hbm,
             o0_hbm, o1_hbm, acc_sh, idx_v, row_v, sem):
    cid = lax.axis_index("c")
    tid = lax.axis_index("s")
    base = tid * ROWS_PER_TILE

    sl0 = pl.ds(base, ROWS_PER_TILE)
    pltpu.sync_copy(z_hbm.at[sl0], acc_sh.at[sl0])
    plsc.subcore_barrier()

    nloop = NCHUNK // NS + (1 if NCHUNK % NS else 0)

    @pl.loop(0, nloop)
    def _(j):
        c = j * NS + tid

        @pl.when(c < NCHUNK)
        def _():
            off = c * CHUNK
            pltpu.sync_copy(src_hbm.at[pl.ds(off, CHUNK)], idx_v.at[0])
            pltpu.sync_copy(dst_hbm.at[pl.ds(off, CHUNK)], idx_v.at[1])

            @pl.when(cid == 0)
            def _():
                pltpu.async_copy(x0_hbm.at[idx_v.at[0]], row_v, sem).wait()

            @pl.when(cid == 1)
            def _():
                pltpu.async_copy(x1_hbm.at[idx_v.at[0]], row_v, sem).wait()

            pltpu.sync_copy(row_v, acc_sh.at[idx_v.at[1]], add=True)

    plsc.subcore_barrier()

    def write_out(rows):
        sl = pl.ds(base, rows)

        @pl.when(cid == 0)
        def _():
            pltpu.sync_copy(acc_sh.at[sl], o0_hbm.at[sl])

        @pl.when(cid == 1)
        def _():
            pltpu.sync_copy(acc_sh.at[sl], o1_hbm.at[sl])

    @pl.when(tid < NS - 1)
    def _():
        write_out(ROWS_PER_TILE)

    @pl.when(tid == NS - 1)
    def _():
        write_out(LAST_ROWS)


@functools.partial(
    pl.kernel,
    out_type=[
        jax.ShapeDtypeStruct((N, H), jnp.float32),
        jax.ShapeDtypeStruct((N, H), jnp.float32),
    ],
    mesh=_sc_mesh(),
    scratch_types=[
        pltpu.VMEM_SHARED((NP, H), jnp.float32),
        pltpu.VMEM((1, CHUNK), jnp.int32),
        pltpu.VMEM((CHUNK, H), jnp.float32),
        pltpu.SemaphoreType.DMA,
    ],
    compiler_params=_sc_params(),
)
def _deg_cnt(dst_hbm, zc_hbm, c0_hbm, c1_hbm, cnt_sh, idx_v, one_v, sem):
    cid = lax.axis_index("c")
    tid = lax.axis_index("s")
    base = tid * ROWS_PER_TILE

    sl0 = pl.ds(base, ROWS_PER_TILE)
    pltpu.sync_copy(zc_hbm.at[sl0], cnt_sh.at[sl0])

    @pl.loop(0, CHUNK)
    def _(r):
        @pl.loop(0, H, step=16)
        def _(cc):
            one_v[r, pl.ds(cc, 16)] = jnp.ones((16,), jnp.float32)

    plsc.subcore_barrier()

    half = NCHUNK // 2

    @pl.loop(0, half // NS + 1)
    def _(j):
        c = j * NS + tid

        @pl.when(c < half)
        def _():
            off = (cid * half + c) * CHUNK
            pltpu.sync_copy(dst_hbm.at[pl.ds(off, CHUNK)], idx_v.at[0])
            pltpu.sync_copy(one_v, cnt_sh.at[idx_v.at[0]], add=True)

    plsc.subcore_barrier()

    def write_out(rows):
        sl = pl.ds(base, rows)

        @pl.when(cid == 0)
        def _():
            pltpu.sync_copy(cnt_sh.at[sl], c0_hbm.at[sl])

        @pl.when(cid == 1)
        def _():
            pltpu.sync_copy(cnt_sh.at[sl], c1_hbm.at[sl])

    @pl.when(tid < NS - 1)
    def _():
        write_out(ROWS_PER_TILE)

    @pl.when(tid == NS - 1)
    def _():
        write_out(LAST_ROWS)


@functools.partial(
    pl.kernel,
    out_type=[
        jax.ShapeDtypeStruct((NDOC, H), jnp.float32),
        jax.ShapeDtypeStruct((NDOC, H), jnp.float32),
    ],
    mesh=_sc_mesh(),
    scratch_types=[
        pltpu.VMEM((DB, DLEN), jnp.int32),
        pltpu.VMEM((DB, DLEN), jnp.float32),
        pltpu.VMEM((DLEN, H), jnp.float32),
        pltpu.VMEM((DB, H), jnp.float32),
        pltpu.SemaphoreType.DMA,
    ],
    compiler_params=_sc_params(),
)
def _doc_pool(y0_hbm, y1_hbm, ids_hbm, w_hbm, e0_hbm, e1_hbm,
              idb, wb, gb, ob, sem):
    cid = lax.axis_index("c")
    tid = lax.axis_index("s")
    base_doc = tid * DOCS_PER_TILE

    @pl.loop(0, DOCS_PER_TILE // DB)
    def _(bb):
        d0 = base_doc + bb * DB
        pltpu.sync_copy(ids_hbm.at[pl.ds(d0, DB)], idb)
        pltpu.sync_copy(w_hbm.at[pl.ds(d0, DB)], wb)

        for d in range(DB):
            @pl.when(cid == 0)
            def _():
                pltpu.async_copy(y0_hbm.at[idb.at[d]], gb, sem).wait()

            @pl.when(cid == 1)
            def _():
                pltpu.async_copy(y1_hbm.at[idb.at[d]], gb, sem).wait()

            def body(l16, accs):
                l0 = l16 * 16
                wvec = wb[d, pl.ds(l0, 16)]
                for i in range(16):
                    wl = wvec[i]
                    accs = tuple(
                        accs[k] + wl * gb[l0 + i, pl.ds(16 * k, 16)]
                        for k in range(8)
                    )
                return accs

            accs = lax.fori_loop(
                0, DLEN // 16, body,
                tuple(jnp.zeros((16,), jnp.float32) for _ in range(8)),
            )
            for k in range(8):
                ob[d, pl.ds(16 * k, 16)] = accs[k]

        @pl.when(cid == 0)
        def _():
            pltpu.sync_copy(ob, e0_hbm.at[pl.ds(d0, DB)])

        @pl.when(cid == 1)
        def _():
            pltpu.sync_copy(ob, e1_hbm.at[pl.ds(d0, DB)])


def _tc_layer(s0, s1, c0, c1, x0, x1, Wl, Wr, b):
    def body(s0_ref, s1_ref, c0_ref, c1_ref, x0_ref, x1_ref, wl_ref, wr_ref,
             b_ref, h_ref, sum_ref, sq_ref):
        i = pl.program_id(0)
        cnt = c0_ref[:, 0:1] + c1_ref[:, 0:1]
        inv = 1.0 / jnp.maximum(cnt, 1.0)
        h = (
            jnp.dot(s0_ref[...] * inv, wl_ref[0:H, :],
                    preferred_element_type=jnp.float32)
            + jnp.dot(s1_ref[...] * inv, wl_ref[H:D, :],
                      preferred_element_type=jnp.float32)
            + jnp.dot(x0_ref[...], wr_ref[0:H, :],
                      preferred_element_type=jnp.float32)
            + jnp.dot(x1_ref[...], wr_ref[H:D, :],
                      preferred_element_type=jnp.float32)
            + b_ref[...]
        )
        h_ref[...] = h

        @pl.when(i == 0)
        def _():
            sum_ref[...] = jnp.zeros_like(sum_ref)
            sq_ref[...] = jnp.zeros_like(sq_ref)

        sum_ref[...] += jnp.sum(h, axis=0, keepdims=True)
        sq_ref[...] += jnp.sum(h * h, axis=0, keepdims=True)

    return pl.pallas_call(
        body,
        grid=(N // BR,),
        in_specs=[
            pl.BlockSpec((BR, H), lambda i: (i, 0)),
            pl.BlockSpec((BR, H), lambda i: (i, 0)),
            pl.BlockSpec((BR, H), lambda i: (i, 0)),
            pl.BlockSpec((BR, H), lambda i: (i, 0)),
            pl.BlockSpec((BR, H), lambda i: (i, 0)),
            pl.BlockSpec((BR, H), lambda i: (i, 0)),
            pl.BlockSpec((D, D), lambda i: (0, 0)),
            pl.BlockSpec((D, D), lambda i: (0, 0)),
            pl.BlockSpec((1, D), lambda i: (0, 0)),
        ],
        out_specs=[
            pl.BlockSpec((BR, D), lambda i: (i, 0)),
            pl.BlockSpec((1, D), lambda i: (0, 0)),
            pl.BlockSpec((1, D), lambda i: (0, 0)),
        ],
        out_shape=[
            jax.ShapeDtypeStruct((N, D), jnp.float32),
            jax.ShapeDtypeStruct((1, D), jnp.float32),
            jax.ShapeDtypeStruct((1, D), jnp.float32),
        ],
    )(s0, s1, c0, c1, x0, x1, Wl, Wr, b.reshape(1, D))


def _tc_bnrelu(h, s, q, g, be):
    def body(h_ref, s_ref, q_ref, g_ref, be_ref, y0_ref, y1_ref):
        m = s_ref[...] / N
        v = q_ref[...] / N - m * m
        scale = g_ref[...] * lax.rsqrt(v + EPS_BN)
        y = jnp.maximum((h_ref[...] - m) * scale + be_ref[...], 0.0)
        y0_ref[...] = y[:, 0:H]
        y1_ref[...] = y[:, H:D]

    return pl.pallas_call(
        body,
        grid=(N // BR,),
        in_specs=[
            pl.BlockSpec((BR, D), lambda i: (i, 0)),
            pl.BlockSpec((1, D), lambda i: (0, 0)),
            pl.BlockSpec((1, D), lambda i: (0, 0)),
            pl.BlockSpec((1, D), lambda i: (0, 0)),
            pl.BlockSpec((1, D), lambda i: (0, 0)),
        ],
        out_specs=[
            pl.BlockSpec((BR, H), lambda i: (i, 0)),
            pl.BlockSpec((BR, H), lambda i: (i, 0)),
        ],
        out_shape=[
            jax.ShapeDtypeStruct((N, H), jnp.float32),
            jax.ShapeDtypeStruct((N, H), jnp.float32),
        ],
    )(h, s, q, g.reshape(1, D), be.reshape(1, D))


def _tc_final(e0, e1, w, Wfc, bfc):
    BF = 1024

    def body(e0_ref, e1_ref, w_ref, wfc_ref, b_ref, o_ref):
        inv = 1.0 / (jnp.sum(w_ref[...], axis=1, keepdims=True) + EPS_POOL)
        o_ref[...] = (
            jnp.dot(e0_ref[...] * inv, wfc_ref[0:H, :],
                    preferred_element_type=jnp.float32)
            + jnp.dot(e1_ref[...] * inv, wfc_ref[H:D, :],
                      preferred_element_type=jnp.float32)
            + b_ref[...]
        )

    return pl.pallas_call(
        body,
        grid=(NDOC // BF,),
        in_specs=[
            pl.BlockSpec((BF, H), lambda i: (i, 0)),
            pl.BlockSpec((BF, H), lambda i: (i, 0)),
            pl.BlockSpec((BF, DLEN), lambda i: (i, 0)),
            pl.BlockSpec((D, NCLS), lambda i: (0, 0)),
            pl.BlockSpec((1, NCLS), lambda i: (0, 0)),
        ],
        out_specs=pl.BlockSpec((BF, NCLS), lambda i: (i, 0)),
        out_shape=jax.ShapeDtypeStruct((NDOC, NCLS), jnp.float32),
    )(e0, e1, w, Wfc, bfc.reshape(1, NCLS))


def kernel(x, edge_index, doc_word_ids, doc_tfidf, Wl1, Wr1, b1, g1, be1,
           Wl2, Wr2, b2, g2, be2, Wfc, bfc):
    src = edge_index[0].astype(jnp.int32)
    dst = edge_index[1].astype(jnp.int32)
    ids = doc_word_ids.astype(jnp.int32)
    x0 = x[:, 0:H]
    x1 = x[:, H:D]
    zf = jnp.zeros((NP, H), jnp.float32)
    c0, c1 = _deg_cnt(dst, zf)
    s0, s1 = _seg_sum(x0, x1, src, dst, zf)
    h1, sm1, sq1 = _tc_layer(s0, s1, c0, c1, x0, x1, Wl1, Wr1, b1)
    y0, y1 = _tc_bnrelu(h1, sm1, sq1, g1, be1)

    t0, t1 = _seg_sum(y0, y1, src, dst, zf)
    h2, sm2, sq2 = _tc_layer(t0, t1, c0, c1, y0, y1, Wl2, Wr2, b2)
    z0, z1 = _tc_bnrelu(h2, sm2, sq2, g2, be2)

    e0, e1 = _doc_pool(z0, z1, ids, doc_tfidf)
    return _tc_final(e0, e1, doc_tfidf, Wfc, bfc)


# speedup vs baseline: 3.7322x; 1.1400x over previous
import dataclasses
import functools

import jax
import jax.numpy as jnp
from jax import lax
from jax.experimental import pallas as pl
from jax.experimental.pallas import tpu as pltpu
from jax.experimental.pallas import tpu_sc as plsc

N = 10000
E = 160000
D = 256
H = 128
NCLS = 16
NDOC = 4096
DLEN = 64

NS = 16
CHUNK = 128
NCHUNK = E // CHUNK
NP = 10240
ROWS_PER_TILE = NP // NS
LAST_ROWS = N - (NS - 1) * ROWS_PER_TILE
DOCS_PER_TILE = NDOC // NS
DB = 8
BR = 1000
EPS_BN = 1e-5
EPS_POOL = 1e-8


def _sc_mesh():
    return plsc.VectorSubcoreMesh(core_axis_name="c", subcore_axis_name="s")


def _sc_params():
    cp = pltpu.CompilerParams()
    if "needs_layout_passes" in pltpu.CompilerParams.__dataclass_fields__:
        cp = dataclasses.replace(cp, needs_layout_passes=False)
    return cp


@functools.partial(
    pl.kernel,
    out_type=[
        jax.ShapeDtypeStruct((N, H), jnp.float32),
        jax.ShapeDtypeStruct((N, H), jnp.float32),
    ],
    mesh=_sc_mesh(),
    scratch_types=[
        pltpu.VMEM_SHARED((NP, H), jnp.float32),
        pltpu.VMEM((2, CHUNK), jnp.int32),
        pltpu.VMEM((CHUNK, H), jnp.float32),
        pltpu.SemaphoreType.DMA,
    ],
    compiler_params=_sc_params(),
)
def _seg_sum(x0_hbm, x1_hbm, src_hbm, dst_hbm, z_hbm,
             o0_hbm, o1_hbm, acc_sh, idx_v, row_v, sem):
    cid = lax.axis_index("c")
    tid = lax.axis_index("s")
    base = tid * ROWS_PER_TILE

    sl0 = pl.ds(base, ROWS_PER_TILE)
    pltpu.sync_copy(z_hbm.at[sl0], acc_sh.at[sl0])
    plsc.subcore_barrier()

    nloop = NCHUNK // NS + (1 if NCHUNK % NS else 0)

    @pl.loop(0, nloop)
    def _(j):
        c = j * NS + tid

        @pl.when(c < NCHUNK)
        def _():
            off = c * CHUNK
            pltpu.sync_copy(src_hbm.at[pl.ds(off, CHUNK)], idx_v.at[0])
            pltpu.sync_copy(dst_hbm.at[pl.ds(off, CHUNK)], idx_v.at[1])

            @pl.when(cid == 0)
            def _():
                pltpu.async_copy(x0_hbm.at[idx_v.at[0]], row_v, sem).wait()

            @pl.when(cid == 1)
            def _():
                pltpu.async_copy(x1_hbm.at[idx_v.at[0]], row_v, sem).wait()

            pltpu.sync_copy(row_v, acc_sh.at[idx_v.at[1]], add=True)

    plsc.subcore_barrier()

    def write_out(rows):
        sl = pl.ds(base, rows)

        @pl.when(cid == 0)
        def _():
            pltpu.sync_copy(acc_sh.at[sl], o0_hbm.at[sl])

        @pl.when(cid == 1)
        def _():
            pltpu.sync_copy(acc_sh.at[sl], o1_hbm.at[sl])

    @pl.when(tid < NS - 1)
    def _():
        write_out(ROWS_PER_TILE)

    @pl.when(tid == NS - 1)
    def _():
        write_out(LAST_ROWS)


@functools.partial(
    pl.kernel,
    out_type=[
        jax.ShapeDtypeStruct((N, H), jnp.float32),
        jax.ShapeDtypeStruct((N, H), jnp.float32),
    ],
    mesh=_sc_mesh(),
    scratch_types=[
        pltpu.VMEM_SHARED((NP, H), jnp.float32),
        pltpu.VMEM((1, CHUNK), jnp.int32),
        pltpu.VMEM((CHUNK, H), jnp.float32),
        pltpu.SemaphoreType.DMA,
    ],
    compiler_params=_sc_params(),
)
def _deg_cnt(dst_hbm, zc_hbm, c0_hbm, c1_hbm, cnt_sh, idx_v, one_v, sem):
    cid = lax.axis_index("c")
    tid = lax.axis_index("s")
    base = tid * ROWS_PER_TILE

    sl0 = pl.ds(base, ROWS_PER_TILE)
    pltpu.sync_copy(zc_hbm.at[sl0], cnt_sh.at[sl0])

    @pl.loop(0, CHUNK)
    def _(r):
        @pl.loop(0, H, step=16)
        def _(cc):
            one_v[r, pl.ds(cc, 16)] = jnp.ones((16,), jnp.float32)

    plsc.subcore_barrier()

    half = NCHUNK // 2

    @pl.loop(0, half // NS + 1)
    def _(j):
        c = j * NS + tid

        @pl.when(c < half)
        def _():
            off = (cid * half + c) * CHUNK
            pltpu.sync_copy(dst_hbm.at[pl.ds(off, CHUNK)], idx_v.at[0])
            pltpu.sync_copy(one_v, cnt_sh.at[idx_v.at[0]], add=True)

    plsc.subcore_barrier()

    def write_out(rows):
        sl = pl.ds(base, rows)

        @pl.when(cid == 0)
        def _():
            pltpu.sync_copy(cnt_sh.at[sl], c0_hbm.at[sl])

        @pl.when(cid == 1)
        def _():
            pltpu.sync_copy(cnt_sh.at[sl], c1_hbm.at[sl])

    @pl.when(tid < NS - 1)
    def _():
        write_out(ROWS_PER_TILE)

    @pl.when(tid == NS - 1)
    def _():
        write_out(LAST_ROWS)


@functools.partial(
    pl.kernel,
    out_type=[
        jax.ShapeDtypeStruct((NDOC, H), jnp.float32),
        jax.ShapeDtypeStruct((NDOC, H), jnp.float32),
    ],
    mesh=_sc_mesh(),
    scratch_types=[
        pltpu.VMEM((DB, DLEN), jnp.int32),
        pltpu.VMEM((DB, DLEN), jnp.float32),
        pltpu.VMEM((DLEN, H), jnp.float32),
        pltpu.VMEM((DLEN, H), jnp.float32),
        pltpu.VMEM((DB, H), jnp.float32),
        pltpu.SemaphoreType.DMA,
        pltpu.SemaphoreType.DMA,
    ],
    compiler_params=_sc_params(),
)
def _doc_pool(y0_hbm, y1_hbm, ids_hbm, w_hbm, e0_hbm, e1_hbm,
              idb, wb, gb0, gb1, ob, sem0, sem1):
    cid = lax.axis_index("c")
    tid = lax.axis_index("s")
    base_doc = tid * DOCS_PER_TILE

    def start_gather(buf, sem, didx):
        @pl.when(cid == 0)
        def _():
            pltpu.async_copy(y0_hbm.at[idb.at[didx]], buf, sem)

        @pl.when(cid == 1)
        def _():
            pltpu.async_copy(y1_hbm.at[idb.at[didx]], buf, sem)

    def wait_gather(buf, sem):
        pltpu.make_async_copy(y0_hbm.at[idb.at[0]], buf, sem).wait()

    @pl.loop(0, DOCS_PER_TILE // DB)
    def _(bb):
        d0 = base_doc + bb * DB
        pltpu.sync_copy(ids_hbm.at[pl.ds(d0, DB)], idb)
        pltpu.sync_copy(w_hbm.at[pl.ds(d0, DB)], wb)

        start_gather(gb0, sem0, 0)
        for d in range(DB):
            cur_buf, cur_sem = (gb0, sem0) if d % 2 == 0 else (gb1, sem1)
            if d + 1 < DB:
                nbuf, nsem = (gb1, sem1) if d % 2 == 0 else (gb0, sem0)
                start_gather(nbuf, nsem, d + 1)
            wait_gather(cur_buf, cur_sem)

            def body(l16, accs, gb=cur_buf):
                l0 = l16 * 16
                wvec = wb[d, pl.ds(l0, 16)]
                for i in range(16):
                    wl = wvec[i]
                    accs = tuple(
                        accs[k] + wl * gb[l0 + i, pl.ds(16 * k, 16)]
                        for k in range(8)
                    )
                return accs

            accs = lax.fori_loop(
                0, DLEN // 16, body,
                tuple(jnp.zeros((16,), jnp.float32) for _ in range(8)),
            )
            for k in range(8):
                ob[d, pl.ds(16 * k, 16)] = accs[k]

        @pl.when(cid == 0)
        def _():
            pltpu.sync_copy(ob, e0_hbm.at[pl.ds(d0, DB)])

        @pl.when(cid == 1)
        def _():
            pltpu.sync_copy(ob, e1_hbm.at[pl.ds(d0, DB)])


def _tc_layer(s0, s1, c0, c1, x0, x1, Wl, Wr, b):
    def body(s0_ref, s1_ref, c0_ref, c1_ref, x0_ref, x1_ref, wl_ref, wr_ref,
             b_ref, h_ref, sum_ref, sq_ref):
        i = pl.program_id(0)
        cnt = c0_ref[:, 0:1] + c1_ref[:, 0:1]
        inv = 1.0 / jnp.maximum(cnt, 1.0)
        h = (
            jnp.dot(s0_ref[...] * inv, wl_ref[0:H, :],
                    preferred_element_type=jnp.float32)
            + jnp.dot(s1_ref[...] * inv, wl_ref[H:D, :],
                      preferred_element_type=jnp.float32)
            + jnp.dot(x0_ref[...], wr_ref[0:H, :],
                      preferred_element_type=jnp.float32)
            + jnp.dot(x1_ref[...], wr_ref[H:D, :],
                      preferred_element_type=jnp.float32)
            + b_ref[...]
        )
        h_ref[...] = h

        @pl.when(i == 0)
        def _():
            sum_ref[...] = jnp.zeros_like(sum_ref)
            sq_ref[...] = jnp.zeros_like(sq_ref)

        sum_ref[...] += jnp.sum(h, axis=0, keepdims=True)
        sq_ref[...] += jnp.sum(h * h, axis=0, keepdims=True)

    return pl.pallas_call(
        body,
        grid=(N // BR,),
        in_specs=[
            pl.BlockSpec((BR, H), lambda i: (i, 0)),
            pl.BlockSpec((BR, H), lambda i: (i, 0)),
            pl.BlockSpec((BR, H), lambda i: (i, 0)),
            pl.BlockSpec((BR, H), lambda i: (i, 0)),
            pl.BlockSpec((BR, H), lambda i: (i, 0)),
            pl.BlockSpec((BR, H), lambda i: (i, 0)),
            pl.BlockSpec((D, D), lambda i: (0, 0)),
            pl.BlockSpec((D, D), lambda i: (0, 0)),
            pl.BlockSpec((1, D), lambda i: (0, 0)),
        ],
        out_specs=[
            pl.BlockSpec((BR, D), lambda i: (i, 0)),
            pl.BlockSpec((1, D), lambda i: (0, 0)),
            pl.BlockSpec((1, D), lambda i: (0, 0)),
        ],
        out_shape=[
            jax.ShapeDtypeStruct((N, D), jnp.float32),
            jax.ShapeDtypeStruct((1, D), jnp.float32),
            jax.ShapeDtypeStruct((1, D), jnp.float32),
        ],
    )(s0, s1, c0, c1, x0, x1, Wl, Wr, b.reshape(1, D))


def _tc_bnrelu(h, s, q, g, be):
    def body(h_ref, s_ref, q_ref, g_ref, be_ref, y0_ref, y1_ref):
        m = s_ref[...] / N
        v = q_ref[...] / N - m * m
        scale = g_ref[...] * lax.rsqrt(v + EPS_BN)
        y = jnp.maximum((h_ref[...] - m) * scale + be_ref[...], 0.0)
        y0_ref[...] = y[:, 0:H]
        y1_ref[...] = y[:, H:D]

    return pl.pallas_call(
        body,
        grid=(N // BR,),
        in_specs=[
            pl.BlockSpec((BR, D), lambda i: (i, 0)),
            pl.BlockSpec((1, D), lambda i: (0, 0)),
            pl.BlockSpec((1, D), lambda i: (0, 0)),
            pl.BlockSpec((1, D), lambda i: (0, 0)),
            pl.BlockSpec((1, D), lambda i: (0, 0)),
        ],
        out_specs=[
            pl.BlockSpec((BR, H), lambda i: (i, 0)),
            pl.BlockSpec((BR, H), lambda i: (i, 0)),
        ],
        out_shape=[
            jax.ShapeDtypeStruct((N, H), jnp.float32),
            jax.ShapeDtypeStruct((N, H), jnp.float32),
        ],
    )(h, s, q, g.reshape(1, D), be.reshape(1, D))


def _tc_final(e0, e1, w, Wfc, bfc):
    BF = 1024

    def body(e0_ref, e1_ref, w_ref, wfc_ref, b_ref, o_ref):
        inv = 1.0 / (jnp.sum(w_ref[...], axis=1, keepdims=True) + EPS_POOL)
        o_ref[...] = (
            jnp.dot(e0_ref[...] * inv, wfc_ref[0:H, :],
                    preferred_element_type=jnp.float32)
            + jnp.dot(e1_ref[...] * inv, wfc_ref[H:D, :],
                      preferred_element_type=jnp.float32)
            + b_ref[...]
        )

    return pl.pallas_call(
        body,
        grid=(NDOC // BF,),
        in_specs=[
            pl.BlockSpec((BF, H), lambda i: (i, 0)),
            pl.BlockSpec((BF, H), lambda i: (i, 0)),
            pl.BlockSpec((BF, DLEN), lambda i: (i, 0)),
            pl.BlockSpec((D, NCLS), lambda i: (0, 0)),
            pl.BlockSpec((1, NCLS), lambda i: (0, 0)),
        ],
        out_specs=pl.BlockSpec((BF, NCLS), lambda i: (i, 0)),
        out_shape=jax.ShapeDtypeStruct((NDOC, NCLS), jnp.float32),
    )(e0, e1, w, Wfc, bfc.reshape(1, NCLS))


def kernel(x, edge_index, doc_word_ids, doc_tfidf, Wl1, Wr1, b1, g1, be1,
           Wl2, Wr2, b2, g2, be2, Wfc, bfc):
    src = edge_index[0].astype(jnp.int32)
    dst = edge_index[1].astype(jnp.int32)
    ids = doc_word_ids.astype(jnp.int32)
    x0 = x[:, 0:H]
    x1 = x[:, H:D]
    zf = jnp.zeros((NP, H), jnp.float32)
    c0, c1 = _deg_cnt(dst, zf)
    s0, s1 = _seg_sum(x0, x1, src, dst, zf)
    h1, sm1, sq1 = _tc_layer(s0, s1, c0, c1, x0, x1, Wl1, Wr1, b1)
    y0, y1 = _tc_bnrelu(h1, sm1, sq1, g1, be1)

    t0, t1 = _seg_sum(y0, y1, src, dst, zf)
    h2, sm2, sq2 = _tc_layer(t0, t1, c0, c1, y0, y1, Wl2, Wr2, b2)
    z0, z1 = _tc_bnrelu(h2, sm2, sq2, g2, be2)

    e0, e1 = _doc_pool(z0, z1, ids, doc_tfidf)
    return _tc_final(e0, e1, doc_tfidf, Wfc, bfc)


# speedup vs baseline: 4.7190x; 1.2644x over previous
import dataclasses
import functools

import jax
import jax.numpy as jnp
from jax import lax
from jax.experimental import pallas as pl
from jax.experimental.pallas import tpu as pltpu
from jax.experimental.pallas import tpu_sc as plsc

N = 10000
E = 160000
D = 256
H = 128
NCLS = 16
NDOC = 4096
DLEN = 64

NS = 16
CHUNK = 128
NCHUNK = E // CHUNK
NP = 10240
ROWS_PER_TILE = NP // NS
LAST_ROWS = N - (NS - 1) * ROWS_PER_TILE
DOCS_PER_TILE = NDOC // NS
DB = 8
BR = 1000
EPS_BN = 1e-5
EPS_POOL = 1e-8


def _sc_mesh():
    return plsc.VectorSubcoreMesh(core_axis_name="c", subcore_axis_name="s")


def _sc_params():
    cp = pltpu.CompilerParams()
    if "needs_layout_passes" in pltpu.CompilerParams.__dataclass_fields__:
        cp = dataclasses.replace(cp, needs_layout_passes=False)
    return cp


@functools.partial(
    pl.kernel,
    out_type=[
        jax.ShapeDtypeStruct((N, H), jnp.float32),
        jax.ShapeDtypeStruct((N, H), jnp.float32),
    ],
    mesh=_sc_mesh(),
    scratch_types=[
        pltpu.VMEM_SHARED((NP, H), jnp.float32),
        pltpu.VMEM((2, CHUNK), jnp.int32),
        pltpu.VMEM((2, CHUNK), jnp.int32),
        pltpu.VMEM((CHUNK, H), jnp.float32),
        pltpu.VMEM((CHUNK, H), jnp.float32),
        pltpu.SemaphoreType.DMA,
        pltpu.SemaphoreType.DMA,
        pltpu.SemaphoreType.DMA,
        pltpu.SemaphoreType.DMA,
    ],
    compiler_params=_sc_params(),
)
def _seg_sum(x0_hbm, x1_hbm, ei_hbm, z_hbm,
             o0_hbm, o1_hbm, acc_sh, ia, ib, ra, rb, gs0, gs1, ss0, ss1):
    cid = lax.axis_index("c")
    tid = lax.axis_index("s")
    base = tid * ROWS_PER_TILE

    sl0 = pl.ds(base, ROWS_PER_TILE)
    pltpu.sync_copy(z_hbm.at[sl0], acc_sh.at[sl0])
    plsc.subcore_barrier()

    def start_gather(idxb, buf, sem):
        @pl.when(cid == 0)
        def _():
            pltpu.async_copy(x0_hbm.at[idxb.at[0]], buf, sem)

        @pl.when(cid == 1)
        def _():
            pltpu.async_copy(x1_hbm.at[idxb.at[0]], buf, sem)

    def wait_dma(buf, sem):
        pltpu.make_async_copy(z_hbm.at[pl.ds(0, CHUNK)], buf, sem).wait()

    npair = (NCHUNK // NS + 1 + 1) // 2

    @pl.loop(0, npair)
    def _(jj):
        c0 = (2 * jj) * NS + tid
        c1 = (2 * jj + 1) * NS + tid

        @pl.when(c0 < NCHUNK)
        def _():
            pltpu.sync_copy(ei_hbm.at[:, pl.ds(c0 * CHUNK, CHUNK)], ia)
            start_gather(ia, ra, gs0)

        @pl.when(c1 < NCHUNK)
        def _():
            pltpu.sync_copy(ei_hbm.at[:, pl.ds(c1 * CHUNK, CHUNK)], ib)
            start_gather(ib, rb, gs1)

        @pl.when(c0 < NCHUNK)
        def _():
            wait_dma(ra, gs0)
            pltpu.async_copy(ra, acc_sh.at[ia.at[1]], ss0, add=True)

        @pl.when(c1 < NCHUNK)
        def _():
            wait_dma(rb, gs1)
            pltpu.async_copy(rb, acc_sh.at[ib.at[1]], ss1, add=True)

        @pl.when(c0 < NCHUNK)
        def _():
            wait_dma(ra, ss0)

        @pl.when(c1 < NCHUNK)
        def _():
            wait_dma(rb, ss1)

    plsc.subcore_barrier()

    def write_out(rows):
        sl = pl.ds(base, rows)

        @pl.when(cid == 0)
        def _():
            pltpu.sync_copy(acc_sh.at[sl], o0_hbm.at[sl])

        @pl.when(cid == 1)
        def _():
            pltpu.sync_copy(acc_sh.at[sl], o1_hbm.at[sl])

    @pl.when(tid < NS - 1)
    def _():
        write_out(ROWS_PER_TILE)

    @pl.when(tid == NS - 1)
    def _():
        write_out(LAST_ROWS)


@functools.partial(
    pl.kernel,
    out_type=[
        jax.ShapeDtypeStruct((N, H), jnp.float32),
        jax.ShapeDtypeStruct((N, H), jnp.float32),
    ],
    mesh=_sc_mesh(),
    scratch_types=[
        pltpu.VMEM_SHARED((NP, H), jnp.float32),
        pltpu.VMEM((1, CHUNK), jnp.int32),
        pltpu.VMEM((CHUNK, H), jnp.float32),
        pltpu.SemaphoreType.DMA,
    ],
    compiler_params=_sc_params(),
)
def _deg_cnt(dst_hbm, zc_hbm, c0_hbm, c1_hbm, cnt_sh, idx_v, one_v, sem):
    cid = lax.axis_index("c")
    tid = lax.axis_index("s")
    base = tid * ROWS_PER_TILE

    sl0 = pl.ds(base, ROWS_PER_TILE)
    pltpu.sync_copy(zc_hbm.at[sl0], cnt_sh.at[sl0])

    @pl.loop(0, CHUNK)
    def _(r):
        @pl.loop(0, H, step=16)
        def _(cc):
            one_v[r, pl.ds(cc, 16)] = jnp.ones((16,), jnp.float32)

    plsc.subcore_barrier()

    half = NCHUNK // 2

    @pl.loop(0, half // NS + 1)
    def _(j):
        c = j * NS + tid

        @pl.when(c < half)
        def _():
            off = (cid * half + c) * CHUNK
            pltpu.sync_copy(dst_hbm.at[pl.ds(off, CHUNK)], idx_v.at[0])
            pltpu.sync_copy(one_v, cnt_sh.at[idx_v.at[0]], add=True)

    plsc.subcore_barrier()

    def write_out(rows):
        sl = pl.ds(base, rows)

        @pl.when(cid == 0)
        def _():
            pltpu.sync_copy(cnt_sh.at[sl], c0_hbm.at[sl])

        @pl.when(cid == 1)
        def _():
            pltpu.sync_copy(cnt_sh.at[sl], c1_hbm.at[sl])

    @pl.when(tid < NS - 1)
    def _():
        write_out(ROWS_PER_TILE)

    @pl.when(tid == NS - 1)
    def _():
        write_out(LAST_ROWS)


@functools.partial(
    pl.kernel,
    out_type=[
        jax.ShapeDtypeStruct((NDOC, H), jnp.float32),
        jax.ShapeDtypeStruct((NDOC, H), jnp.float32),
    ],
    mesh=_sc_mesh(),
    scratch_types=[
        pltpu.VMEM((DB, DLEN), jnp.int32),
        pltpu.VMEM((DB, DLEN), jnp.float32),
        pltpu.VMEM((DLEN, H), jnp.float32),
        pltpu.VMEM((DLEN, H), jnp.float32),
        pltpu.VMEM((DB, H), jnp.float32),
        pltpu.SemaphoreType.DMA,
        pltpu.SemaphoreType.DMA,
    ],
    compiler_params=_sc_params(),
)
def _doc_pool(y0_hbm, y1_hbm, ids_hbm, w_hbm, e0_hbm, e1_hbm,
              idb, wb, gb0, gb1, ob, sem0, sem1):
    cid = lax.axis_index("c")
    tid = lax.axis_index("s")
    base_doc = tid * DOCS_PER_TILE

    def start_gather(buf, sem, didx):
        @pl.when(cid == 0)
        def _():
            pltpu.async_copy(y0_hbm.at[idb.at[didx]], buf, sem)

        @pl.when(cid == 1)
        def _():
            pltpu.async_copy(y1_hbm.at[idb.at[didx]], buf, sem)

    def wait_gather(buf, sem):
        pltpu.make_async_copy(y0_hbm.at[idb.at[0]], buf, sem).wait()

    @pl.loop(0, DOCS_PER_TILE // DB)
    def _(bb):
        d0 = base_doc + bb * DB
        pltpu.sync_copy(ids_hbm.at[pl.ds(d0, DB)], idb)
        pltpu.sync_copy(w_hbm.at[pl.ds(d0, DB)], wb)

        start_gather(gb0, sem0, 0)
        for d in range(DB):
            cur_buf, cur_sem = (gb0, sem0) if d % 2 == 0 else (gb1, sem1)
            if d + 1 < DB:
                nbuf, nsem = (gb1, sem1) if d % 2 == 0 else (gb0, sem0)
                start_gather(nbuf, nsem, d + 1)
            wait_gather(cur_buf, cur_sem)

            def body(l16, accs, gb=cur_buf):
                l0 = l16 * 16
                wvec = wb[d, pl.ds(l0, 16)]
                for i in range(16):
                    wl = wvec[i]
                    accs = tuple(
                        accs[k] + wl * gb[l0 + i, pl.ds(16 * k, 16)]
                        for k in range(8)
                    )
                return accs

            accs = lax.fori_loop(
                0, DLEN // 16, body,
                tuple(jnp.zeros((16,), jnp.float32) for _ in range(8)),
            )
            for k in range(8):
                ob[d, pl.ds(16 * k, 16)] = accs[k]

        @pl.when(cid == 0)
        def _():
            pltpu.sync_copy(ob, e0_hbm.at[pl.ds(d0, DB)])

        @pl.when(cid == 1)
        def _():
            pltpu.sync_copy(ob, e1_hbm.at[pl.ds(d0, DB)])


def _tc_layer(s0, s1, c0, c1, x0, x1, Wl, Wr, b):
    def body(s0_ref, s1_ref, c0_ref, c1_ref, x0_ref, x1_ref, wl_ref, wr_ref,
             b_ref, h_ref, sum_ref, sq_ref):
        i = pl.program_id(0)
        cnt = c0_ref[:, 0:1] + c1_ref[:, 0:1]
        inv = 1.0 / jnp.maximum(cnt, 1.0)
        h = (
            jnp.dot(s0_ref[...] * inv, wl_ref[0:H, :],
                    preferred_element_type=jnp.float32)
            + jnp.dot(s1_ref[...] * inv, wl_ref[H:D, :],
                      preferred_element_type=jnp.float32)
            + jnp.dot(x0_ref[...], wr_ref[0:H, :],
                      preferred_element_type=jnp.float32)
            + jnp.dot(x1_ref[...], wr_ref[H:D, :],
                      preferred_element_type=jnp.float32)
            + b_ref[...]
        )
        h_ref[...] = h

        @pl.when(i == 0)
        def _():
            sum_ref[...] = jnp.zeros_like(sum_ref)
            sq_ref[...] = jnp.zeros_like(sq_ref)

        sum_ref[...] += jnp.sum(h, axis=0, keepdims=True)
        sq_ref[...] += jnp.sum(h * h, axis=0, keepdims=True)

    return pl.pallas_call(
        body,
        grid=(N // BR,),
        in_specs=[
            pl.BlockSpec((BR, H), lambda i: (i, 0)),
            pl.BlockSpec((BR, H), lambda i: (i, 0)),
            pl.BlockSpec((BR, H), lambda i: (i, 0)),
            pl.BlockSpec((BR, H), lambda i: (i, 0)),
            pl.BlockSpec((BR, H), lambda i: (i, 0)),
            pl.BlockSpec((BR, H), lambda i: (i, 0)),
            pl.BlockSpec((D, D), lambda i: (0, 0)),
            pl.BlockSpec((D, D), lambda i: (0, 0)),
            pl.BlockSpec((1, D), lambda i: (0, 0)),
        ],
        out_specs=[
            pl.BlockSpec((BR, D), lambda i: (i, 0)),
            pl.BlockSpec((1, D), lambda i: (0, 0)),
            pl.BlockSpec((1, D), lambda i: (0, 0)),
        ],
        out_shape=[
            jax.ShapeDtypeStruct((N, D), jnp.float32),
            jax.ShapeDtypeStruct((1, D), jnp.float32),
            jax.ShapeDtypeStruct((1, D), jnp.float32),
        ],
    )(s0, s1, c0, c1, x0, x1, Wl, Wr, b.reshape(1, D))


def _tc_bnrelu(h, s, q, g, be):
    def body(h_ref, s_ref, q_ref, g_ref, be_ref, y0_ref, y1_ref):
        m = s_ref[...] / N
        v = q_ref[...] / N - m * m
        scale = g_ref[...] * lax.rsqrt(v + EPS_BN)
        y = jnp.maximum((h_ref[...] - m) * scale + be_ref[...], 0.0)
        y0_ref[...] = y[:, 0:H]
        y1_ref[...] = y[:, H:D]

    return pl.pallas_call(
        body,
        grid=(N // BR,),
        in_specs=[
            pl.BlockSpec((BR, D), lambda i: (i, 0)),
            pl.BlockSpec((1, D), lambda i: (0, 0)),
            pl.BlockSpec((1, D), lambda i: (0, 0)),
            pl.BlockSpec((1, D), lambda i: (0, 0)),
            pl.BlockSpec((1, D), lambda i: (0, 0)),
        ],
        out_specs=[
            pl.BlockSpec((BR, H), lambda i: (i, 0)),
            pl.BlockSpec((BR, H), lambda i: (i, 0)),
        ],
        out_shape=[
            jax.ShapeDtypeStruct((N, H), jnp.float32),
            jax.ShapeDtypeStruct((N, H), jnp.float32),
        ],
    )(h, s, q, g.reshape(1, D), be.reshape(1, D))


def _tc_final(e0, e1, w, Wfc, bfc):
    BF = 1024

    def body(e0_ref, e1_ref, w_ref, wfc_ref, b_ref, o_ref):
        inv = 1.0 / (jnp.sum(w_ref[...], axis=1, keepdims=True) + EPS_POOL)
        o_ref[...] = (
            jnp.dot(e0_ref[...] * inv, wfc_ref[0:H, :],
                    preferred_element_type=jnp.float32)
            + jnp.dot(e1_ref[...] * inv, wfc_ref[H:D, :],
                      preferred_element_type=jnp.float32)
            + b_ref[...]
        )

    return pl.pallas_call(
        body,
        grid=(NDOC // BF,),
        in_specs=[
            pl.BlockSpec((BF, H), lambda i: (i, 0)),
            pl.BlockSpec((BF, H), lambda i: (i, 0)),
            pl.BlockSpec((BF, DLEN), lambda i: (i, 0)),
            pl.BlockSpec((D, NCLS), lambda i: (0, 0)),
            pl.BlockSpec((1, NCLS), lambda i: (0, 0)),
        ],
        out_specs=pl.BlockSpec((BF, NCLS), lambda i: (i, 0)),
        out_shape=jax.ShapeDtypeStruct((NDOC, NCLS), jnp.float32),
    )(e0, e1, w, Wfc, bfc.reshape(1, NCLS))


def kernel(x, edge_index, doc_word_ids, doc_tfidf, Wl1, Wr1, b1, g1, be1,
           Wl2, Wr2, b2, g2, be2, Wfc, bfc):
    ei = edge_index.astype(jnp.int32)
    dst = ei[1]
    ids = doc_word_ids.astype(jnp.int32)
    x0 = x[:, 0:H]
    x1 = x[:, H:D]
    zf = jnp.zeros((NP, H), jnp.float32)
    c0, c1 = _deg_cnt(dst, zf)
    s0, s1 = _seg_sum(x0, x1, ei, zf)
    h1, sm1, sq1 = _tc_layer(s0, s1, c0, c1, x0, x1, Wl1, Wr1, b1)
    y0, y1 = _tc_bnrelu(h1, sm1, sq1, g1, be1)

    t0, t1 = _seg_sum(y0, y1, ei, zf)
    h2, sm2, sq2 = _tc_layer(t0, t1, c0, c1, y0, y1, Wl2, Wr2, b2)
    z0, z1 = _tc_bnrelu(h2, sm2, sq2, g2, be2)

    e0, e1 = _doc_pool(z0, z1, ids, doc_tfidf)
    return _tc_final(e0, e1, doc_tfidf, Wfc, bfc)
